# baseline (speedup 1.0000x reference)
import jax
import jax.numpy as jnp
from jax import lax
from jax.experimental import pallas as pl
from jax.experimental.pallas import tpu as pltpu
from jax.experimental.pallas import tpu_sc as plsc

NUM_PERM = 8
C = 8
N = 1024
D = 768

_ROWS = NUM_PERM * C * N
_BLOCK_ROWS = 4096

_NC = 2
_NS = 16
_NW = _NC * _NS
_ACHUNK = (NUM_PERM * N) // _NW


def _copy_body(in_ref, out_ref):
    out_ref[...] = in_ref[...]


_tc_copy = pl.pallas_call(
    _copy_body,
    grid=(_ROWS // _BLOCK_ROWS,),
    in_specs=[pl.BlockSpec((_BLOCK_ROWS, D), lambda i: (i, 0))],
    out_specs=pl.BlockSpec((_BLOCK_ROWS, D), lambda i: (i, 0)),
    out_shape=jax.ShapeDtypeStruct((_ROWS, D), jnp.float32),
    compiler_params=pltpu.CompilerParams(dimension_semantics=("parallel",)),
)


def _sc_idx_body(argsort_hbm, perm_hbm, asort_v, perm_v):
    cid = lax.axis_index("c")
    sid = lax.axis_index("s")
    wid = sid * _NC + cid

    abase = wid * _ACHUNK
    row_off = lax.rem(abase, N)
    for v in range(_ACHUNK // 16):
        asort_v[pl.ds(v * 16, 16)] = lax.iota(jnp.int32, 16) + (row_off + v * 16)
    pltpu.sync_copy(asort_v, argsort_hbm.at[pl.ds(abase, _ACHUNK)])

    @pl.when(wid == 0)
    def _():
        perm_v[...] = lax.iota(jnp.int32, 16)
        pltpu.sync_copy(perm_v, perm_hbm)


_sc_idx = pl.kernel(
    _sc_idx_body,
    out_type=(
        jax.ShapeDtypeStruct((NUM_PERM * N,), jnp.int32),
        jax.ShapeDtypeStruct((16,), jnp.int32),
    ),
    mesh=plsc.VectorSubcoreMesh(core_axis_name="c", subcore_axis_name="s"),
    scratch_types=[
        pltpu.VMEM((_ACHUNK,), jnp.int32),
        pltpu.VMEM((16,), jnp.int32),
    ],
    compiler_params=pltpu.CompilerParams(skip_device_barrier=True),
)


def kernel(patches):
    argsort_flat, perm16 = _sc_idx()
    aug = _tc_copy(patches.reshape(_ROWS, D))
    return (
        aug.reshape(NUM_PERM, C, N, D),
        argsort_flat.reshape(NUM_PERM, N),
        perm16[:NUM_PERM],
    )
